# baseline (speedup 1.0000x reference)
import functools

import jax
import jax.numpy as jnp
from jax import lax
from jax.experimental import pallas as pl
from jax.experimental.pallas import tpu as pltpu
from jax.experimental.pallas import tpu_sc as plsc

_NC = 2
_NS = 16
_NW = _NC * _NS
_L = 16

_N = 32 * 512 * 512
_NPW = _N // _NW
_CHUNK = 16384
_NCHUNKS = _NPW // _CHUNK
_U = 4


def _sc_body(sal_hbm, fix_hbm, out_hbm, sbuf, fbuf, part,
             sem_s0, sem_s1, sem_f0, sem_f1):
    wid = lax.axis_index("s") * _NC + lax.axis_index("c")
    base = wid * _NPW
    sems = (sem_s0, sem_s1)
    semf = (sem_f0, sem_f1)

    def copies(c, b):
        off = base + c * _CHUNK
        return (
            pltpu.make_async_copy(sal_hbm.at[pl.ds(off, _CHUNK)],
                                  sbuf.at[b], sems[b]),
            pltpu.make_async_copy(fix_hbm.at[pl.ds(off, _CHUNK)],
                                  fbuf.at[b], semf[b]),
        )

    for b in range(2):
        for cp in copies(b, b):
            cp.start()

    zero = jnp.zeros((_L,), jnp.float32)
    accs0 = (zero, zero, zero, zero)

    def chunk_compute(b, accs):
        def step(j, accs):
            a_sum, a_sq, a_m, a_c = accs
            for u in range(_U):
                idx = pl.ds((j * _U + u) * _L, _L)
                v = sbuf[b, idx]
                f = fbuf[b, idx]
                m = f > 0.1
                a_sum = a_sum + v
                a_sq = a_sq + v * v
                a_m = a_m + jnp.where(m, v, 0.0)
                a_c = a_c + jnp.where(m, 1.0, 0.0)
            return (a_sum, a_sq, a_m, a_c)

        return lax.fori_loop(0, _CHUNK // (_L * _U), step, accs)

    def outer(g, accs):
        for b in range(2):
            c = g * 2 + b
            for cp in copies(c, b):
                cp.wait()
            accs = chunk_compute(b, accs)

            @pl.when(c + 2 < _NCHUNKS)
            def _():
                for cp in copies(c + 2, b):
                    cp.start()

        return accs

    a_sum, a_sq, a_m, a_c = lax.fori_loop(0, _NCHUNKS // 2, outer, accs0)

    part[0, :] = a_sum
    part[1, :] = a_sq
    part[2, :] = a_m
    part[3, :] = a_c
    pltpu.sync_copy(part, out_hbm.at[wid])


_sc_reduce = functools.partial(
    pl.kernel,
    mesh=plsc.VectorSubcoreMesh(core_axis_name="c", subcore_axis_name="s"),
    out_type=jax.ShapeDtypeStruct((_NW, 4, _L), jnp.float32),
    scratch_types=[
        pltpu.VMEM((2, _CHUNK), jnp.float32),
        pltpu.VMEM((2, _CHUNK), jnp.float32),
        pltpu.VMEM((4, _L), jnp.float32),
        pltpu.SemaphoreType.DMA,
        pltpu.SemaphoreType.DMA,
        pltpu.SemaphoreType.DMA,
        pltpu.SemaphoreType.DMA,
    ],
)(_sc_body)


def kernel(sal_map, fix):
    n = sal_map.size
    partials = _sc_reduce(sal_map.reshape(-1), fix.reshape(-1))
    sums = jnp.sum(partials, axis=(0, 2))
    ssum, ssq, msum, cnt = sums[0], sums[1], sums[2], sums[3]
    nf = jnp.float32(n)
    mean = ssum / nf
    var = (ssq - nf * mean * mean) / (nf - 1.0)
    std = jnp.sqrt(var)
    return (msum - cnt * mean) / (std * cnt)
